# baseline (speedup 1.0000x reference)
import functools
import math

import jax
import jax.numpy as jnp
from jax import lax
from jax.experimental import pallas as pl
from jax.experimental.pallas import tpu as pltpu
from jax.experimental.pallas import tpu_sc as plsc

jax.config.update("jax_enable_x64", True)

_LANES = 128
_NC, _NS = 2, 16
_CHUNK = 128


def _srl(v, n):
    return lax.shift_right_logical(v, jnp.int32(n))


def _hash_body(max_ngram, n_heads, sh_ref, mc_ref, rk_ref, mod_ref, off_ref,
               out_ref):
    digits = []
    for k in range(max_ngram):
        ids = sh_ref[k]
        c0, c1, c2 = mc_ref[k, 0], mc_ref[k, 1], mc_ref[k, 2]
        p0 = ids * c0
        p1 = ids * c1
        p2 = ids * c2
        d0 = p0 & 0xFFFF
        t1 = _srl(p0, 16) + (p1 & 0xFFFF)
        d1 = t1 & 0xFFFF
        t2 = _srl(p1, 16) + (p2 & 0xFFFF) + _srl(t1, 16)
        d2 = t2 & 0xFFFF
        d3 = _srl(t2, 16) + _srl(p2, 16)
        digits.append((d0, d1, d2, d3))
    mix = digits[0]
    for n in range(2, max_ngram + 1):
        mix = tuple(a ^ b for a, b in zip(mix, digits[n - 1]))
        for h in range(n_heads):
            head = (n - 2) * n_heads + h
            s = None
            for j, d in enumerate(mix):
                lo = d & 0xFF
                hi = _srl(d, 8)
                term = lo * rk_ref[head, 2 * j] + hi * rk_ref[head, 2 * j + 1]
                s = term if s is None else s + term
            out_ref[head] = jnp.mod(s, mod_ref[head]) + off_ref[head]


def _compute_flat_idx(input_ids, hash_mults, hash_mods, offsets):
    b, t = input_ids.shape
    max_ngram = int(hash_mults.shape[0])
    total_heads = int(hash_mods.shape[0])
    n_heads = total_heads // (max_ngram - 1)
    nr = (b * t) // _LANES

    ids32 = input_ids.astype(jnp.int32)
    shifts = [ids32]
    for k in range(1, max_ngram):
        shifts.append(jnp.pad(ids32, ((0, 0), (k, 0)))[:, :t])
    sh = jnp.stack(shifts).reshape(max_ngram, nr, _LANES)

    m64 = hash_mults.astype(jnp.int64)
    mc = jnp.stack([m64 & 0xFFFF, (m64 >> 16) & 0xFFFF, (m64 >> 32) & 0xFFFF],
                   axis=1).astype(jnp.int32)
    pow8 = jnp.asarray([1 << (8 * j) for j in range(8)], dtype=jnp.int64)
    rk = jnp.mod(pow8[None, :], hash_mods[:, None]).astype(jnp.int32)
    mods32 = hash_mods.astype(jnp.int32)
    offs32 = offsets.astype(jnp.int32)

    smem = pl.BlockSpec(memory_space=pltpu.SMEM)
    hashes = pl.pallas_call(
        functools.partial(_hash_body, max_ngram, n_heads),
        out_shape=jax.ShapeDtypeStruct((total_heads, nr, _LANES), jnp.int32),
        in_specs=[pl.BlockSpec(memory_space=pltpu.VMEM),
                  smem, smem, smem, smem],
    )(sh, mc, rk, mods32, offs32)
    return hashes.reshape(total_heads, b * t).T.reshape(-1)


def _gather_body(n_chunks, table_hbm, idx_hbm, out_hbm, idx_v, rows_v, sem):
    wid = lax.axis_index("s") * _NC + lax.axis_index("c")
    base = wid * (n_chunks * _CHUNK)
    for c in range(n_chunks):
        off = base + c * _CHUNK
        pltpu.sync_copy(idx_hbm.at[pl.ds(off, _CHUNK)], idx_v)
        pltpu.async_copy(table_hbm.at[idx_v], rows_v, sem).wait()
        pltpu.sync_copy(rows_v, out_hbm.at[pl.ds(off, _CHUNK)])


def _sc_gather(emb_table, flat_idx):
    n_idx = int(flat_idx.shape[0])
    d = int(emb_table.shape[1])
    nw = _NC * _NS
    n_chunks = n_idx // (nw * _CHUNK)
    mesh = plsc.VectorSubcoreMesh(core_axis_name="c", subcore_axis_name="s")
    k = pl.kernel(
        functools.partial(_gather_body, n_chunks),
        out_type=jax.ShapeDtypeStruct((n_idx, d), jnp.float32),
        mesh=mesh,
        scratch_types=[
            pltpu.VMEM((_CHUNK,), jnp.int32),
            pltpu.VMEM((_CHUNK, d), jnp.float32),
            pltpu.SemaphoreType.DMA,
        ],
        compiler_params=pltpu.CompilerParams(use_tc_tiling_on_sc=False),
    )
    return k(emb_table, flat_idx)


def _proj_body(hidden, eps, emb_ref, x_ref, wvt_ref, wkt_ref, qw_ref, kw_ref,
               out_ref):
    emb = emb_ref[...]
    value = jnp.dot(emb, wvt_ref[...], preferred_element_type=jnp.float32)
    key_v = jnp.dot(emb, wkt_ref[...], preferred_element_type=jnp.float32)
    x = x_ref[...]
    q = x * lax.rsqrt(jnp.mean(x * x, axis=-1, keepdims=True) + eps)
    q = q * qw_ref[...]
    kn = key_v * lax.rsqrt(jnp.mean(key_v * key_v, axis=-1, keepdims=True)
                           + eps)
    kn = kn * kw_ref[...]
    gate = jax.nn.sigmoid(jnp.sum(q * kn, axis=-1, keepdims=True)
                          / math.sqrt(float(hidden)))
    out_ref[...] = gate * value


def _proj(emb, x2d, Wv, Wk, qw, kw):
    n, e = emb.shape
    hidden = x2d.shape[1]
    blk = 512
    grid = (n // blk,)
    return pl.pallas_call(
        functools.partial(_proj_body, hidden, 1e-6),
        grid=grid,
        in_specs=[
            pl.BlockSpec((blk, e), lambda i: (i, i * 0)),
            pl.BlockSpec((blk, hidden), lambda i: (i, i * 0)),
            pl.BlockSpec((e, hidden), lambda i: (i * 0, i * 0)),
            pl.BlockSpec((e, hidden), lambda i: (i * 0, i * 0)),
            pl.BlockSpec((1, hidden), lambda i: (i * 0, i * 0)),
            pl.BlockSpec((1, hidden), lambda i: (i * 0, i * 0)),
        ],
        out_specs=pl.BlockSpec((blk, hidden), lambda i: (i, i * 0)),
        out_shape=jax.ShapeDtypeStruct((n, hidden), jnp.float32),
    )(emb, x2d, Wv.T, Wk.T, qw.reshape(1, -1), kw.reshape(1, -1))


def kernel(x, input_ids, emb_table, Wv, Wk, key_norm_w, query_norm_w,
           hash_mults, hash_mods, offsets):
    b, t, hidden = x.shape
    flat_idx = _compute_flat_idx(input_ids, hash_mults, hash_mods, offsets)
    rows = _sc_gather(emb_table.astype(jnp.float32), flat_idx)
    emb = rows.reshape(b * t, -1)
    out = _proj(emb, x.reshape(b * t, hidden).astype(jnp.float32),
                Wv.astype(jnp.float32), Wk.astype(jnp.float32),
                key_norm_w.astype(jnp.float32),
                query_norm_w.astype(jnp.float32))
    return out.reshape(b, t, hidden)
